# baseline (speedup 1.0000x reference)
import jax
import jax.numpy as jnp
from jax.experimental import pallas as pl

K = 16
HI = jax.lax.Precision.HIGHEST


def _lrelu(x):
    return jnp.where(x >= 0, x, 0.2 * x)


def kernel(feat, W1, g1, be1, W2, b2, W3, g3, be3, W4, g4, be4, W5, g5, be5, W6, b6):
    B, n, d = feat.shape
    X = feat[0]
    xb = X.astype(jnp.bfloat16)
    dot = jnp.einsum('nd,md->nm', xb, xb, preferred_element_type=jnp.float32)
    xx = jnp.sum(X ** 2, axis=1)
    pairwise = -xx[:, None] - (-2.0 * dot) - xx[None, :]
    _, idx = jax.lax.top_k(pairwise, K + 1)
    idx = idx[:, 1:]

    scale = 1.0 / jnp.sqrt(1.0 + 1e-5)
    a1 = (g1 * scale)
    W1a = W1[:, :d]
    W1b = W1[:, d:]
    u = jnp.einsum('nd,od->no', X, W1a, precision=HI)
    v = jnp.einsum('nd,od->no', X, W1b - W1a, precision=HI)

    g = u[idx.reshape(-1)]
    e = g.reshape(n, K, -1) + v[:, None, :]
    e = _lrelu(e * a1[None, None, :] + be1[None, None, :])
    h2 = jnp.einsum('nkc,oc->nko', e, W2, precision=HI) + b2[None, None, :]
    feat1 = jnp.max(h2, axis=1)
    fmax = jnp.max(feat1, axis=0)
    favg = jnp.mean(feat1, axis=0)

    cat = jnp.concatenate([X, feat1,
                           jnp.broadcast_to(fmax[None, :], feat1.shape),
                           jnp.broadcast_to(favg[None, :], feat1.shape)], axis=1)
    y = _lrelu(jnp.einsum('nc,oc->no', cat, W3, precision=HI) * (g3 * scale) + be3)
    y = _lrelu(jnp.einsum('nc,oc->no', y, W4, precision=HI) * (g4 * scale) + be4)
    y = _lrelu(jnp.einsum('nc,oc->no', y, W5, precision=HI) * (g5 * scale) + be5)
    y = jnp.einsum('nc,oc->no', y, W6, precision=HI) + b6[None, :]
    return y[None, :, :]


# speedup vs baseline: 51.6668x; 51.6668x over previous
import functools

import jax
import jax.numpy as jnp
from jax import lax
from jax.experimental import pallas as pl
from jax.experimental.pallas import tpu as pltpu
from jax.experimental.pallas import tpu_sc as plsc

KNN = 16
N = 10000
NP = 10240
NA = 5120
NB = N - NA
D_IN = 128
BIG = 1e30
HI = jax.lax.Precision.HIGHEST

R1 = 256


def _lrelu(x):
    return jnp.where(x >= 0, x, 0.2 * x)


def _k1_body(off, xb_ref, xbt_ref, xxr_ref, xxc_ref, idx_ref):
    i = pl.program_id(0)
    dot = jnp.dot(xb_ref[...], xbt_ref[...], preferred_element_type=jnp.float32)
    inner = -2.0 * dot
    dmat = -xxc_ref[...] - inner - xxr_ref[...]
    ci = lax.broadcasted_iota(jnp.int32, (R1, NP), 1)
    grow = off + i * R1 + lax.broadcasted_iota(jnp.int32, (R1, 1), 0)
    dmat = jnp.where(ci == grow, -BIG, dmat)

    ninf = jnp.full((R1, 128), -jnp.inf, jnp.float32)
    zero = jnp.zeros((R1, 128), jnp.int32)
    sv1 = [ninf] * 4
    sv2 = [ninf] * 4
    si1 = [zero] * 4
    si2 = [zero] * 4
    for c in range(NP // 128):
        st = c % 4
        v = dmat[:, c * 128:(c + 1) * 128]
        b1 = v > sv1[st]
        b2 = v > sv2[st]
        n2 = jnp.where(b1, sv1[st], jnp.where(b2, v, sv2[st]))
        m2 = jnp.where(b1, si1[st], jnp.where(b2, c, si2[st]))
        sv1[st] = jnp.maximum(v, sv1[st])
        si1[st] = jnp.where(b1, c, si1[st])
        sv2[st] = n2
        si2[st] = m2

    lane = lax.broadcasted_iota(jnp.int32, (R1, 128), 1)
    av = jnp.concatenate(sv1 + sv2, axis=1)
    gidx = jnp.concatenate([i_ * 128 + lane for i_ in si1 + si2], axis=1)
    cols = []
    for _ in range(KNN):
        m = jnp.max(av, axis=1, keepdims=True)
        am = jnp.min(jnp.where(av == m, gidx, NP), axis=1)
        cols.append(am)
        av = jnp.where(gidx == am[:, None], -jnp.inf, av)
    idx_ref[...] = jnp.stack(cols, axis=1)


def _run_k1(xb_half, xbt, xxr_half, xxc, off):
    rows = xb_half.shape[0]
    grid = (rows // R1,)
    return pl.pallas_call(
        functools.partial(_k1_body, off),
        grid=grid,
        in_specs=[
            pl.BlockSpec((R1, D_IN), lambda i: (i, 0)),
            pl.BlockSpec((D_IN, NP), lambda i: (0, 0)),
            pl.BlockSpec((R1, 1), lambda i: (i, 0)),
            pl.BlockSpec((1, NP), lambda i: (0, 0)),
        ],
        out_specs=pl.BlockSpec((R1, KNN), lambda i: (i, 0)),
        out_shape=jax.ShapeDtypeStruct((rows, KNN), jnp.int32),
    )(xb_half, xbt, xxr_half, xxc)


def _k0_body(xt_ref, xx_ref):
    xt = xt_ref[...]
    xx_ref[...] = jnp.sum(xt * xt, axis=0, keepdims=True)


def _run_k0(xt):
    return pl.pallas_call(
        _k0_body,
        in_specs=[pl.BlockSpec((D_IN, NP), lambda: (0, 0))],
        out_specs=pl.BlockSpec((1, NP), lambda: (0, 0)),
        out_shape=jax.ShapeDtypeStruct((1, NP), jnp.float32),
        grid=(),
    )(xt)


def _k5_body(x_ref, w1at_ref, w1vt_ref, u_ref, v_ref):
    u_ref[...] = jnp.dot(x_ref[...], w1at_ref[...],
                         preferred_element_type=jnp.float32, precision=HI)
    v_ref[...] = jnp.dot(x_ref[...], w1vt_ref[...],
                         preferred_element_type=jnp.float32, precision=HI)


def _run_k5(xp, w1at, w1vt):
    grid = (NP // 1024,)
    return pl.pallas_call(
        _k5_body,
        grid=grid,
        in_specs=[
            pl.BlockSpec((1024, D_IN), lambda i: (i, 0)),
            pl.BlockSpec((D_IN, 256), lambda i: (0, 0)),
            pl.BlockSpec((D_IN, 256), lambda i: (0, 0)),
        ],
        out_specs=[
            pl.BlockSpec((1024, 256), lambda i: (i, 0)),
            pl.BlockSpec((1024, 256), lambda i: (i, 0)),
        ],
        out_shape=[
            jax.ShapeDtypeStruct((NP, 256), jnp.float32),
            jax.ShapeDtypeStruct((NP, 256), jnp.float32),
        ],
    )(xp, w1at, w1vt)


_NW = 32


def _k2_body(epw, chunk, u_hbm, idx_hbm, g_hbm, idx_v, rows_v, sem):
    wid = lax.axis_index("s") * 2 + lax.axis_index("c")
    base = wid * epw

    def step(t, _):
        off = base + t * chunk
        pltpu.sync_copy(idx_hbm.at[pl.ds(off, chunk)], idx_v)
        pltpu.async_copy(u_hbm.at[idx_v], rows_v, sem).wait()
        pltpu.sync_copy(rows_v, g_hbm.at[pl.ds(off, chunk)])
        return 0

    lax.fori_loop(0, epw // chunk, step, 0)


def _run_k2(u, idxf, chunk):
    ne = idxf.shape[0]
    epw = ne // _NW
    mesh = plsc.VectorSubcoreMesh(core_axis_name="c", subcore_axis_name="s")
    fn = functools.partial(
        pl.kernel, mesh=mesh,
        out_type=jax.ShapeDtypeStruct((ne, 256), jnp.float32),
        scratch_types=[
            pltpu.VMEM((chunk,), jnp.int32),
            pltpu.VMEM((chunk, 256), jnp.float32),
            pltpu.SemaphoreType.DMA,
        ],
    )(functools.partial(_k2_body, epw, chunk))
    return fn(u, idxf)


def _k3_body(r3, g_ref, v_ref, a1_ref, b1_ref, w2t_ref, b2_ref,
             feat1_ref, gmax_ref, gsum_ref):
    i = pl.program_id(0)
    v = v_ref[...]
    vrep = jnp.broadcast_to(v.reshape(r3, 1, 256), (r3, KNN, 256))
    vrep = vrep.reshape(r3 * KNN, 256)
    h = g_ref[...] + vrep
    h = _lrelu(h * a1_ref[...] + b1_ref[...])
    h2 = jnp.dot(h, w2t_ref[...],
                 preferred_element_type=jnp.float32) + b2_ref[...]
    f1 = jnp.max(h2.reshape(r3, KNN, 128), axis=1)
    feat1_ref[...] = f1
    tm = jnp.max(f1.reshape(r3 // 8, 8, 128), axis=0)
    ts = jnp.sum(f1.reshape(r3 // 8, 8, 128), axis=0)
    gmax_ref[...] = jnp.where(i == 0, tm, jnp.maximum(gmax_ref[...], tm))
    gsum_ref[...] = jnp.where(i == 0, ts, gsum_ref[...] + ts)


def _run_k3(g, v, a1, b1, w2t, b2, r3):
    rows = v.shape[0]
    grid = (rows // r3,)
    return pl.pallas_call(
        functools.partial(_k3_body, r3),
        grid=grid,
        in_specs=[
            pl.BlockSpec((r3 * KNN, 256), lambda i: (i, 0)),
            pl.BlockSpec((r3, 256), lambda i: (i, 0)),
            pl.BlockSpec((1, 256), lambda i: (0, 0)),
            pl.BlockSpec((1, 256), lambda i: (0, 0)),
            pl.BlockSpec((256, 128), lambda i: (0, 0)),
            pl.BlockSpec((1, 128), lambda i: (0, 0)),
        ],
        out_specs=[
            pl.BlockSpec((r3, 128), lambda i: (i, 0)),
            pl.BlockSpec((8, 128), lambda i: (0, 0)),
            pl.BlockSpec((8, 128), lambda i: (0, 0)),
        ],
        out_shape=[
            jax.ShapeDtypeStruct((rows, 128), jnp.float32),
            jax.ShapeDtypeStruct((8, 128), jnp.float32),
            jax.ShapeDtypeStruct((8, 128), jnp.float32),
        ],
    )(g, v, a1, b1, w2t, b2)


def _k4_body(r4, x_ref, f1_ref, gma_ref, gmb_ref, gsa_ref, gsb_ref,
             w3t_ref, a3_ref, b3_ref, w4t_ref, a4_ref, b4_ref,
             w5t_ref, a5_ref, b5_ref, w6t_ref, b6_ref, out_ref):
    gm = jnp.maximum(gma_ref[...], gmb_ref[...])
    fm = jnp.max(gm, axis=0, keepdims=True)
    fa = (jnp.sum(gsa_ref[...], axis=0, keepdims=True) +
          jnp.sum(gsb_ref[...], axis=0, keepdims=True)) * (1.0 / N)
    cat = jnp.concatenate(
        [x_ref[...], f1_ref[...],
         jnp.broadcast_to(fm, (r4, 128)), jnp.broadcast_to(fa, (r4, 128))],
        axis=1)
    y = _lrelu(jnp.dot(cat, w3t_ref[...],
                       preferred_element_type=jnp.float32) * a3_ref[...] + b3_ref[...])
    y = _lrelu(jnp.dot(y, w4t_ref[...],
                       preferred_element_type=jnp.float32) * a4_ref[...] + b4_ref[...])
    y = _lrelu(jnp.dot(y, w5t_ref[...],
                       preferred_element_type=jnp.float32) * a5_ref[...] + b5_ref[...])
    out_ref[...] = jnp.dot(y, w6t_ref[...],
                           preferred_element_type=jnp.float32) + b6_ref[...]


def _run_k4(x, f1, gma, gmb, gsa, gsb,
            w3t, a3, b3, w4t, a4, b4, w5t, a5, b5, w6t, b6, r4):
    rows = x.shape[0]
    grid = (rows // r4,)
    row = lambda c: pl.BlockSpec((1, c), lambda i: (0, 0))
    small = lambda: pl.BlockSpec((8, 128), lambda i: (0, 0))
    return pl.pallas_call(
        functools.partial(_k4_body, r4),
        grid=grid,
        in_specs=[
            pl.BlockSpec((r4, 128), lambda i: (i, 0)),
            pl.BlockSpec((r4, 128), lambda i: (i, 0)),
            small(), small(), small(), small(),
            pl.BlockSpec((512, 384), lambda i: (0, 0)), row(384), row(384),
            pl.BlockSpec((384, 256), lambda i: (0, 0)), row(256), row(256),
            pl.BlockSpec((256, 128), lambda i: (0, 0)), row(128), row(128),
            pl.BlockSpec((128, 128), lambda i: (0, 0)), row(128),
        ],
        out_specs=pl.BlockSpec((r4, 128), lambda i: (i, 0)),
        out_shape=jax.ShapeDtypeStruct((rows, 128), jnp.float32),
    )(x, f1, gma, gmb, gsa, gsb,
      w3t, a3, b3, w4t, a4, b4, w5t, a5, b5, w6t, b6)


def kernel(feat, W1, g1, be1, W2, b2, W3, g3, be3, W4, g4, be4, W5, g5, be5,
           W6, b6):
    X = feat[0]
    xp = jnp.zeros((NP, D_IN), jnp.float32).at[:N].set(X)
    xb = xp.astype(jnp.bfloat16)
    xbt = xb.T

    xxc = _run_k0(xp.T)
    pad_mask = (jnp.arange(NP) >= N)[None, :]
    xxc = jnp.where(pad_mask, BIG, xxc)
    xxr = xxc.reshape(NP, 1)

    scale = 1.0 / jnp.sqrt(jnp.float32(1.0 + 1e-5))
    W1a = W1[:, :D_IN]
    W1v = W1[:, D_IN:] - W1a
    u, v = _run_k5(xp, W1a.T, W1v.T)

    idxA = _run_k1(xb[:NA], xbt, xxr[:NA], xxc, 0)
    gA = _run_k2(u, idxA.reshape(-1), 320)
    idxB = _run_k1(xb[NA:], xbt, xxr[NA:], xxc, NA)
    gB = _run_k2(u, idxB[:NB].reshape(-1), 488)

    a1 = (g1 * scale).reshape(1, 256)
    b1 = be1.reshape(1, 256)
    w2t = W2.T
    b2r = b2.reshape(1, 128)
    f1A, gmA, gsA = _run_k3(gA, v[:NA], a1, b1, w2t, b2r, 320)
    f1B, gmB, gsB = _run_k3(gB, v[NA:N], a1, b1, w2t, b2r, 488)

    mlp = (W3.T, (g3 * scale).reshape(1, 384), be3.reshape(1, 384),
           W4.T, (g4 * scale).reshape(1, 256), be4.reshape(1, 256),
           W5.T, (g5 * scale).reshape(1, 128), be5.reshape(1, 128),
           W6.T, b6.reshape(1, 128))
    outA = _run_k4(X[:NA], f1A, gmA, gmB, gsA, gsB, *mlp, r4=320)
    outB = _run_k4(X[NA:], f1B, gmA, gmB, gsA, gsB, *mlp, r4=488)
    return jnp.concatenate([outA, outB], axis=0)[None]


# speedup vs baseline: 58.7798x; 1.1377x over previous
import functools

import jax
import jax.numpy as jnp
from jax import lax
from jax.experimental import pallas as pl
from jax.experimental.pallas import tpu as pltpu
from jax.experimental.pallas import tpu_sc as plsc

KNN = 16
N = 10000
NP = 10240
NA = 5120
NB = N - NA
D_IN = 128
BIG = 1e30
HI = jax.lax.Precision.HIGHEST

R1 = 256


def _lrelu(x):
    return jnp.where(x >= 0, x, 0.2 * x)


def _k1_body(off, xb_ref, xbt_ref, xxr_ref, xxc_ref, idx_ref):
    i = pl.program_id(0)
    dot = jnp.dot(xb_ref[...], xbt_ref[...], preferred_element_type=jnp.float32)
    inner = -2.0 * dot
    dmat = -xxc_ref[...] - inner - xxr_ref[...]
    ci = lax.broadcasted_iota(jnp.int32, (R1, NP), 1)
    grow = off + i * R1 + lax.broadcasted_iota(jnp.int32, (R1, 1), 0)
    dmat = jnp.where(ci == grow, -BIG, dmat)

    ninf = jnp.full((R1, 128), -jnp.inf, jnp.float32)
    zero = jnp.zeros((R1, 128), jnp.int32)
    sv1 = [ninf] * 2
    sv2 = [ninf] * 2
    si1 = [zero] * 2
    si2 = [zero] * 2
    for c in range(NP // 128):
        st = c % 2
        v = dmat[:, c * 128:(c + 1) * 128]
        b1 = v > sv1[st]
        b2 = v > sv2[st]
        n2 = jnp.where(b1, sv1[st], jnp.where(b2, v, sv2[st]))
        m2 = jnp.where(b1, si1[st], jnp.where(b2, c, si2[st]))
        sv1[st] = jnp.maximum(v, sv1[st])
        si1[st] = jnp.where(b1, c, si1[st])
        sv2[st] = n2
        si2[st] = m2

    lane = lax.broadcasted_iota(jnp.int32, (R1, 128), 1)
    av = jnp.concatenate(sv1 + sv2, axis=1)
    gidx = jnp.concatenate([i_ * 128 + lane for i_ in si1 + si2], axis=1)
    cols = []
    for _ in range(KNN):
        m = jnp.max(av, axis=1, keepdims=True)
        am = jnp.min(jnp.where(av == m, gidx, NP), axis=1)
        cols.append(am)
        av = jnp.where(gidx == am[:, None], -jnp.inf, av)
    idx_ref[...] = jnp.stack(cols, axis=1)


def _run_k1(xb_half, xbt, xxr_half, xxc, off):
    rows = xb_half.shape[0]
    grid = (rows // R1,)
    return pl.pallas_call(
        functools.partial(_k1_body, off),
        grid=grid,
        in_specs=[
            pl.BlockSpec((R1, D_IN), lambda i: (i, 0)),
            pl.BlockSpec((D_IN, NP), lambda i: (0, 0)),
            pl.BlockSpec((R1, 1), lambda i: (i, 0)),
            pl.BlockSpec((1, NP), lambda i: (0, 0)),
        ],
        out_specs=pl.BlockSpec((R1, KNN), lambda i: (i, 0)),
        out_shape=jax.ShapeDtypeStruct((rows, KNN), jnp.int32),
    )(xb_half, xbt, xxr_half, xxc)


def _k0_body(xt_ref, xx_ref):
    xt = xt_ref[...]
    xx_ref[...] = jnp.sum(xt * xt, axis=0, keepdims=True)


def _run_k0(xt):
    return pl.pallas_call(
        _k0_body,
        in_specs=[pl.BlockSpec((D_IN, NP), lambda: (0, 0))],
        out_specs=pl.BlockSpec((1, NP), lambda: (0, 0)),
        out_shape=jax.ShapeDtypeStruct((1, NP), jnp.float32),
        grid=(),
    )(xt)


def _k5_body(x_ref, w1at_ref, w1vt_ref, u_ref, v_ref):
    u_ref[...] = jnp.dot(x_ref[...], w1at_ref[...],
                         preferred_element_type=jnp.float32, precision=HI)
    v_ref[...] = jnp.dot(x_ref[...], w1vt_ref[...],
                         preferred_element_type=jnp.float32, precision=HI)


def _run_k5(xp, w1at, w1vt):
    grid = (NP // 1024,)
    return pl.pallas_call(
        _k5_body,
        grid=grid,
        in_specs=[
            pl.BlockSpec((1024, D_IN), lambda i: (i, 0)),
            pl.BlockSpec((D_IN, 256), lambda i: (0, 0)),
            pl.BlockSpec((D_IN, 256), lambda i: (0, 0)),
        ],
        out_specs=[
            pl.BlockSpec((1024, 256), lambda i: (i, 0)),
            pl.BlockSpec((1024, 256), lambda i: (i, 0)),
        ],
        out_shape=[
            jax.ShapeDtypeStruct((NP, 256), jnp.float32),
            jax.ShapeDtypeStruct((NP, 256), jnp.float32),
        ],
    )(xp, w1at, w1vt)


_NW = 32


def _k2_body(epw, chunk, u_hbm, idx_hbm, g_hbm, idx_v, rows_v, sem):
    wid = lax.axis_index("s") * 2 + lax.axis_index("c")
    base = wid * epw

    def step(t, _):
        off = base + t * chunk
        pltpu.sync_copy(idx_hbm.at[pl.ds(off, chunk)], idx_v)
        pltpu.async_copy(u_hbm.at[idx_v], rows_v, sem).wait()
        pltpu.sync_copy(rows_v, g_hbm.at[pl.ds(off, chunk)])
        return 0

    lax.fori_loop(0, epw // chunk, step, 0)


def _run_k2(u, idxf, chunk):
    ne = idxf.shape[0]
    epw = ne // _NW
    mesh = plsc.VectorSubcoreMesh(core_axis_name="c", subcore_axis_name="s")
    fn = functools.partial(
        pl.kernel, mesh=mesh,
        out_type=jax.ShapeDtypeStruct((ne, 256), jnp.float32),
        scratch_types=[
            pltpu.VMEM((chunk,), jnp.int32),
            pltpu.VMEM((chunk, 256), jnp.float32),
            pltpu.SemaphoreType.DMA,
        ],
    )(functools.partial(_k2_body, epw, chunk))
    return fn(u, idxf)


def _k3_body(r3, g_ref, v_ref, a1_ref, b1_ref, w2t_ref, b2_ref,
             feat1_ref, gmax_ref, gsum_ref):
    i = pl.program_id(0)
    v = v_ref[...]
    vrep = jnp.broadcast_to(v.reshape(r3, 1, 256), (r3, KNN, 256))
    vrep = vrep.reshape(r3 * KNN, 256)
    h = g_ref[...] + vrep
    h = _lrelu(h * a1_ref[...] + b1_ref[...])
    h2 = jnp.dot(h, w2t_ref[...],
                 preferred_element_type=jnp.float32) + b2_ref[...]
    f1 = jnp.max(h2.reshape(r3, KNN, 128), axis=1)
    feat1_ref[...] = f1
    tm = jnp.max(f1.reshape(r3 // 8, 8, 128), axis=0)
    ts = jnp.sum(f1.reshape(r3 // 8, 8, 128), axis=0)
    gmax_ref[...] = jnp.where(i == 0, tm, jnp.maximum(gmax_ref[...], tm))
    gsum_ref[...] = jnp.where(i == 0, ts, gsum_ref[...] + ts)


def _run_k3(g, v, a1, b1, w2t, b2, r3):
    rows = v.shape[0]
    grid = (rows // r3,)
    return pl.pallas_call(
        functools.partial(_k3_body, r3),
        grid=grid,
        in_specs=[
            pl.BlockSpec((r3 * KNN, 256), lambda i: (i, 0)),
            pl.BlockSpec((r3, 256), lambda i: (i, 0)),
            pl.BlockSpec((1, 256), lambda i: (0, 0)),
            pl.BlockSpec((1, 256), lambda i: (0, 0)),
            pl.BlockSpec((256, 128), lambda i: (0, 0)),
            pl.BlockSpec((1, 128), lambda i: (0, 0)),
        ],
        out_specs=[
            pl.BlockSpec((r3, 128), lambda i: (i, 0)),
            pl.BlockSpec((8, 128), lambda i: (0, 0)),
            pl.BlockSpec((8, 128), lambda i: (0, 0)),
        ],
        out_shape=[
            jax.ShapeDtypeStruct((rows, 128), jnp.float32),
            jax.ShapeDtypeStruct((8, 128), jnp.float32),
            jax.ShapeDtypeStruct((8, 128), jnp.float32),
        ],
    )(g, v, a1, b1, w2t, b2)


def _k4_body(r4, x_ref, f1_ref, gma_ref, gmb_ref, gsa_ref, gsb_ref,
             w3t_ref, a3_ref, b3_ref, w4t_ref, a4_ref, b4_ref,
             w5t_ref, a5_ref, b5_ref, w6t_ref, b6_ref, out_ref):
    gm = jnp.maximum(gma_ref[...], gmb_ref[...])
    fm = jnp.max(gm, axis=0, keepdims=True)
    fa = (jnp.sum(gsa_ref[...], axis=0, keepdims=True) +
          jnp.sum(gsb_ref[...], axis=0, keepdims=True)) * (1.0 / N)
    cat = jnp.concatenate(
        [x_ref[...], f1_ref[...],
         jnp.broadcast_to(fm, (r4, 128)), jnp.broadcast_to(fa, (r4, 128))],
        axis=1)
    y = _lrelu(jnp.dot(cat, w3t_ref[...],
                       preferred_element_type=jnp.float32) * a3_ref[...] + b3_ref[...])
    y = _lrelu(jnp.dot(y, w4t_ref[...],
                       preferred_element_type=jnp.float32) * a4_ref[...] + b4_ref[...])
    y = _lrelu(jnp.dot(y, w5t_ref[...],
                       preferred_element_type=jnp.float32) * a5_ref[...] + b5_ref[...])
    out_ref[...] = jnp.dot(y, w6t_ref[...],
                           preferred_element_type=jnp.float32) + b6_ref[...]


def _run_k4(x, f1, gma, gmb, gsa, gsb,
            w3t, a3, b3, w4t, a4, b4, w5t, a5, b5, w6t, b6, r4):
    rows = x.shape[0]
    grid = (rows // r4,)
    row = lambda c: pl.BlockSpec((1, c), lambda i: (0, 0))
    small = lambda: pl.BlockSpec((8, 128), lambda i: (0, 0))
    return pl.pallas_call(
        functools.partial(_k4_body, r4),
        grid=grid,
        in_specs=[
            pl.BlockSpec((r4, 128), lambda i: (i, 0)),
            pl.BlockSpec((r4, 128), lambda i: (i, 0)),
            small(), small(), small(), small(),
            pl.BlockSpec((512, 384), lambda i: (0, 0)), row(384), row(384),
            pl.BlockSpec((384, 256), lambda i: (0, 0)), row(256), row(256),
            pl.BlockSpec((256, 128), lambda i: (0, 0)), row(128), row(128),
            pl.BlockSpec((128, 128), lambda i: (0, 0)), row(128),
        ],
        out_specs=pl.BlockSpec((r4, 128), lambda i: (i, 0)),
        out_shape=jax.ShapeDtypeStruct((rows, 128), jnp.float32),
    )(x, f1, gma, gmb, gsa, gsb,
      w3t, a3, b3, w4t, a4, b4, w5t, a5, b5, w6t, b6)


def kernel(feat, W1, g1, be1, W2, b2, W3, g3, be3, W4, g4, be4, W5, g5, be5,
           W6, b6):
    X = feat[0]
    xp = jnp.zeros((NP, D_IN), jnp.float32).at[:N].set(X)
    xb = xp.astype(jnp.bfloat16)
    xbt = xb.T

    xxc = _run_k0(xp.T)
    pad_mask = (jnp.arange(NP) >= N)[None, :]
    xxc = jnp.where(pad_mask, BIG, xxc)
    xxr = xxc.reshape(NP, 1)

    scale = 1.0 / jnp.sqrt(jnp.float32(1.0 + 1e-5))
    W1a = W1[:, :D_IN]
    W1v = W1[:, D_IN:] - W1a
    u, v = _run_k5(xp, W1a.T, W1v.T)

    idxA = _run_k1(xb[:NA], xbt, xxr[:NA], xxc, 0)
    gA = _run_k2(u, idxA.reshape(-1), 320)
    idxB = _run_k1(xb[NA:], xbt, xxr[NA:], xxc, NA)
    gB = _run_k2(u, idxB[:NB].reshape(-1), 488)

    a1 = (g1 * scale).reshape(1, 256)
    b1 = be1.reshape(1, 256)
    w2t = W2.T
    b2r = b2.reshape(1, 128)
    f1A, gmA, gsA = _run_k3(gA, v[:NA], a1, b1, w2t, b2r, 320)
    f1B, gmB, gsB = _run_k3(gB, v[NA:N], a1, b1, w2t, b2r, 488)

    mlp = (W3.T, (g3 * scale).reshape(1, 384), be3.reshape(1, 384),
           W4.T, (g4 * scale).reshape(1, 256), be4.reshape(1, 256),
           W5.T, (g5 * scale).reshape(1, 128), be5.reshape(1, 128),
           W6.T, b6.reshape(1, 128))
    outA = _run_k4(X[:NA], f1A, gmA, gmB, gsA, gsB, *mlp, r4=320)
    outB = _run_k4(X[NA:], f1B, gmA, gmB, gsA, gsB, *mlp, r4=488)
    return jnp.concatenate([outA, outB], axis=0)[None]
